# baseline (speedup 1.0000x reference)
import functools

import jax
import jax.numpy as jnp
from jax import lax
from jax.experimental import pallas as pl
from jax.experimental.pallas import tpu as pltpu
from jax.experimental.pallas import tpu_sc as plsc

B = 1024
L = 256
NP = 10
D = 200
NT = 4

NC = 2
NS = 16
NW = NC * NS
EPW = B // NW

ROW = (NP + L) * D
PAT = L * D
PRO = NP * D

HALF0 = 26608
HALF1 = ROW - HALF0
RING = 4
AHEAD = 2


def _sc_body(task_id_hbm, patch_hbm, prompt_hbm, out_hbm,
             tid_v, b0, b1, b2, b3, in0, in1, in2, in3, o0, o1, o2, o3):
    bufs = (b0, b1, b2, b3)
    in_sems = (in0, in1, in2, in3)
    out_sems = (o0, o1, o2, o3)

    wid = lax.axis_index("s") * NC + lax.axis_index("c")
    base = wid * EPW

    pltpu.sync_copy(task_id_hbm.at[pl.ds(base, EPW)], tid_v)
    vecs = [tid_v[pl.ds(g * 16, 16)] for g in range(EPW // 16)]

    nchunks = 2 * EPW
    in_h = [None] * nchunks
    out_h = [None] * nchunks

    def start_in(c):
        b = c % RING
        e, h = divmod(c, 2)
        i = base + e
        copies = []
        if h == 0:
            tid = vecs[e // 16][e % 16]
            copies.append(pltpu.async_copy(
                prompt_hbm.at[pl.ds(tid * PRO, PRO)],
                bufs[b].at[pl.ds(0, PRO)], in_sems[b]))
            copies.append(pltpu.async_copy(
                patch_hbm.at[pl.ds(i * PAT, HALF0 - PRO)],
                bufs[b].at[pl.ds(PRO, HALF0 - PRO)], in_sems[b]))
        else:
            copies.append(pltpu.async_copy(
                patch_hbm.at[pl.ds(i * PAT + (HALF0 - PRO), HALF1)],
                bufs[b].at[pl.ds(0, HALF1)], in_sems[b]))
        return copies

    def start_out(c):
        b = c % RING
        e, h = divmod(c, 2)
        off = (base + e) * ROW + h * HALF0
        sz = HALF0 if h == 0 else HALF1
        return pltpu.async_copy(
            bufs[b].at[pl.ds(0, sz)],
            out_hbm.at[pl.ds(off, sz)], out_sems[b])

    for c in range(nchunks):
        if c >= RING:
            out_h[c - RING].wait()
        in_h[c] = start_in(c)
        if c >= AHEAD:
            j = c - AHEAD
            for hdl in in_h[j]:
                hdl.wait()
            out_h[j] = start_out(j)
    for j in range(nchunks - AHEAD, nchunks):
        for hdl in in_h[j]:
            hdl.wait()
        out_h[j] = start_out(j)
    for j in range(nchunks - RING, nchunks):
        out_h[j].wait()


@jax.jit
def _sc_concat(task_id, patch_embeddings, prompt_tokens):
    mesh = plsc.VectorSubcoreMesh(core_axis_name="c", subcore_axis_name="s")
    fn = functools.partial(
        pl.kernel,
        mesh=mesh,
        out_type=jax.ShapeDtypeStruct((B * ROW,), jnp.float32),
        scratch_types=[
            pltpu.VMEM((EPW,), jnp.int32),
            pltpu.VMEM((HALF0,), jnp.float32),
            pltpu.VMEM((HALF0,), jnp.float32),
            pltpu.VMEM((HALF0,), jnp.float32),
            pltpu.VMEM((HALF0,), jnp.float32),
            pltpu.SemaphoreType.DMA,
            pltpu.SemaphoreType.DMA,
            pltpu.SemaphoreType.DMA,
            pltpu.SemaphoreType.DMA,
            pltpu.SemaphoreType.DMA,
            pltpu.SemaphoreType.DMA,
            pltpu.SemaphoreType.DMA,
            pltpu.SemaphoreType.DMA,
        ],
    )(_sc_body)
    out = fn(task_id,
             patch_embeddings.reshape(B * L * D),
             prompt_tokens.reshape(NT * NP * D))
    return out.reshape(B, NP + L, D)


def kernel(task_id, patch_embeddings, prompt_tokens):
    return _sc_concat(task_id.astype(jnp.int32), patch_embeddings,
                      prompt_tokens)


# speedup vs baseline: 2.0073x; 2.0073x over previous
import functools

import jax
import jax.numpy as jnp
from jax.experimental import pallas as pl
from jax.experimental.pallas import tpu as pltpu

B = 1024
L = 256
NP = 10
D = 200
NT = 4

BT = 8


def _body(sref, patch_ref, prompt_ref, out_ref):
    i = pl.program_id(0)
    out_ref[:, NP:, :] = patch_ref[...]
    for b in range(BT):
        tid = sref[i * BT + b]
        out_ref[b, :NP, :] = prompt_ref[tid]


@jax.jit
def _concat(task_id, patch_embeddings, prompt_tokens):
    grid_spec = pltpu.PrefetchScalarGridSpec(
        num_scalar_prefetch=1,
        grid=(B // BT,),
        in_specs=[
            pl.BlockSpec((BT, L, D), lambda i, s: (i, 0, 0)),
            pl.BlockSpec((NT, NP, D), lambda i, s: (0, 0, 0)),
        ],
        out_specs=pl.BlockSpec((BT, NP + L, D), lambda i, s: (i, 0, 0)),
    )
    fn = pl.pallas_call(
        _body,
        grid_spec=grid_spec,
        out_shape=jax.ShapeDtypeStruct((B, NP + L, D), jnp.float32),
        compiler_params=pltpu.CompilerParams(
            dimension_semantics=("arbitrary",)),
    )
    return fn(task_id, patch_embeddings, prompt_tokens)


def kernel(task_id, patch_embeddings, prompt_tokens):
    return _concat(task_id.astype(jnp.int32), patch_embeddings,
                   prompt_tokens)


# speedup vs baseline: 7.4787x; 3.7257x over previous
import jax
import jax.numpy as jnp
from jax.experimental import pallas as pl
from jax.experimental.pallas import tpu as pltpu

B = 1024
L = 256
NP = 10
D = 200
NT = 4

IB = 128
DB = 8


def _body(tid_ref, patch_ref, prompt_ref, out_ref):
    x = patch_ref[...]
    y = jnp.transpose(x, (2, 1, 0))
    out_ref[NP:, :, :] = y
    tid = tid_ref[0, 0, :]
    tab = prompt_ref[...]
    acc = jnp.zeros((NP, DB, IB), jnp.float32)
    for t in range(NT):
        cand = jnp.transpose(tab[t], (1, 0))[:, :, None]
        mask = (tid == t)[None, None, :]
        acc = jnp.where(mask, cand, acc)
    out_ref[:NP, :, :] = acc


@jax.jit
def _concat(task_id, patch_embeddings, prompt_tokens):
    patch_t = jnp.transpose(patch_embeddings, (0, 2, 1))
    prompt_t = jnp.transpose(prompt_tokens, (0, 2, 1))
    tid2 = task_id.reshape(B // IB, 1, IB)
    fn = pl.pallas_call(
        _body,
        grid=(B // IB, D // DB),
        in_specs=[
            pl.BlockSpec((1, 1, IB), lambda ib, db: (ib, 0, 0)),
            pl.BlockSpec((IB, DB, L), lambda ib, db: (ib, db, 0)),
            pl.BlockSpec((NT, DB, NP), lambda ib, db: (0, db, 0)),
        ],
        out_specs=pl.BlockSpec((NP + L, DB, IB), lambda ib, db: (0, db, ib)),
        out_shape=jax.ShapeDtypeStruct((NP + L, D, B), jnp.float32),
        compiler_params=pltpu.CompilerParams(
            dimension_semantics=("parallel", "parallel")),
    )
    out_t = fn(tid2, patch_t, prompt_t)
    return jnp.transpose(out_t, (2, 0, 1))


def kernel(task_id, patch_embeddings, prompt_tokens):
    return _concat(task_id.astype(jnp.int32), patch_embeddings,
                   prompt_tokens)
